# baseline (speedup 1.0000x reference)
import jax
import jax.numpy as jnp
from jax import lax
from jax.experimental import pallas as pl
from jax.experimental.pallas import tpu as pltpu
from jax.experimental.pallas import tpu_sc as plsc

N = 10000
E = 320000
IN_F = 128
HID = 32
HEADS = 4
OUT_F = 40
ALPHA = 0.2
EPS = 1e-16

NP = 10240
DUMP = N
CHUNK = 128
TILES = 16
WORKERS = 32
CH1 = 158
CH2 = 80
D2P = 40
NP1 = 10112
ROWS_PER = NP // TILES
RP1 = NP1 // TILES


def _mm_body(x_ref, w_ref, o_ref):
    o_ref[...] = jnp.dot(x_ref[...], w_ref[...],
                         preferred_element_type=jnp.float32)


def _tc_matmul(x, w, bn=1000):
    n = x.shape[0]
    k = x.shape[1]
    m = w.shape[1]
    return pl.pallas_call(
        _mm_body,
        grid=(n // bn,),
        in_specs=[pl.BlockSpec((bn, k), lambda i: (i, 0)),
                  pl.BlockSpec((k, m), lambda i: (0, 0))],
        out_specs=pl.BlockSpec((bn, m), lambda i: (i, 0)),
        out_shape=jax.ShapeDtypeStruct((n, m), jnp.float32),
    )(x, w)


def _norm_mm_body(a_ref, d_ref, w_ref, o_ref):
    h = a_ref[...] / (d_ref[...] + EPS)
    h = jnp.where(h > 0, h, jnp.exp(jnp.minimum(h, 0.0)) - 1.0)
    o_ref[...] = jnp.dot(h, w_ref[...], preferred_element_type=jnp.float32)


def _tc_norm_matmul(acc, den, w, bn=1000):
    n, k = acc.shape
    m = w.shape[1]
    return pl.pallas_call(
        _norm_mm_body,
        grid=(n // bn,),
        in_specs=[pl.BlockSpec((bn, k), lambda i: (i, 0)),
                  pl.BlockSpec((bn, k), lambda i: (i, 0)),
                  pl.BlockSpec((k, m), lambda i: (0, 0))],
        out_specs=pl.BlockSpec((bn, m), lambda i: (i, 0)),
        out_shape=jax.ShapeDtypeStruct((n, m), jnp.float32),
    )(acc, den, w)


def _final_body(a0_ref, a1_ref, d0_ref, d1_ref, o_ref):
    den = d0_ref[...] + d1_ref[...] + EPS
    o = (a0_ref[...] + a1_ref[...]) / den
    m = jnp.max(o, axis=1, keepdims=True)
    o = o - m
    lse = jnp.log(jnp.sum(jnp.exp(o), axis=1, keepdims=True))
    o_ref[...] = o - lse


def _tc_final(a0, a1, d0, d1, bn=1000):
    n, m = a0.shape
    return pl.pallas_call(
        _final_body,
        grid=(n // bn,),
        in_specs=[pl.BlockSpec((bn, m), lambda i: (i, 0)),
                  pl.BlockSpec((bn, m), lambda i: (i, 0)),
                  pl.BlockSpec((bn, 1), lambda i: (i, 0)),
                  pl.BlockSpec((bn, 1), lambda i: (i, 0))],
        out_specs=pl.BlockSpec((bn, m), lambda i: (i, 0)),
        out_shape=jax.ShapeDtypeStruct((n, m), jnp.float32),
    )(a0, a1, d0, d1)


_MESH = plsc.VectorSubcoreMesh(core_axis_name="c", subcore_axis_name="s")


def _edge_ex_phase(srcb, dstb, stab, dtab, exbuf, j):
    for i in range(CHUNK // 16):
        si = srcb[j, pl.ds(i * 16, 16)]
        di = dstb[j, pl.ds(i * 16, 16)]
        sval = plsc.load_gather(stab, [si])
        dval = plsc.load_gather(dtab, [di])
        e = sval + dval
        e = jnp.maximum(e, ALPHA * e)
        exbuf[pl.ds(i * 16, 16)] = jnp.exp(e)


def _scale_rows(rowbuf, exbuf, nreg, tail=0, unroll=4):
    lane = lax.iota(jnp.int32, 16)
    tmask = lane < tail
    tcol = lane + nreg * 16

    def body(eu, carry):
        for u in range(unroll):
            e = eu * unroll + u
            ev = jnp.full((16,), 0, jnp.int32) + e
            ex = plsc.load_gather(exbuf, [ev])
            for k in range(nreg):
                v = rowbuf[e, pl.ds(k * 16, 16)]
                rowbuf[e, pl.ds(k * 16, 16)] = v * ex
            if tail:
                v = plsc.load_gather(rowbuf, [ev, tcol], mask=tmask)
                plsc.store_scatter(rowbuf, [ev, tcol], v * ex, mask=tmask)
        return carry
    lax.fori_loop(0, CHUNK // unroll, body, 0)


def _zero_buf(buf, ncol_reg, tail=0):
    zv = jnp.zeros((16,), jnp.float32)
    lane = lax.iota(jnp.int32, 16)
    tmask = lane < tail
    tcol = lane + ncol_reg * 16

    def zb(e, carry):
        for k in range(ncol_reg):
            buf[e, pl.ds(k * 16, 16)] = zv
        if tail:
            ev = jnp.full((16,), 0, jnp.int32) + e
            plsc.store_scatter(buf, [ev, tcol], zv, mask=tmask)
        return carry
    lax.fori_loop(0, CHUNK, zb, 0)


def _sync_chunks(nchunks, tab, srcb, dstb, stab, dtab, row, ex,
                 ACCv, DENv, gs, nreg, tail=0):
    def chunk_body(j, carry):
        cp = pltpu.async_copy(tab.at[srcb.at[j]], row, gs)
        _edge_ex_phase(srcb, dstb, stab, dtab, ex, j)
        cp.wait()
        _scale_rows(row, ex, nreg, tail)
        pltpu.sync_copy(row, ACCv.at[dstb.at[j]], add=True)
        pltpu.sync_copy(ex, DENv.at[dstb.at[j]], add=True)
        return carry
    lax.fori_loop(0, nchunks, chunk_body, 0)


def _pipe_chunks(npairs, tab, srcb, dstb, stab, dtab,
                 rowA, rowB, exA, exB, ACCv, DENv, gs, ss, nreg, tail=0):
    pltpu.make_async_copy(tab.at[srcb.at[0]], rowA, gs).start()

    def pair(j2, carry):
        jA = 2 * j2
        jB = jA + 1
        _edge_ex_phase(srcb, dstb, stab, dtab, exA, jA)
        pltpu.make_async_copy(tab.at[srcb.at[jA]], rowA, gs).wait()
        _scale_rows(rowA, exA, nreg, tail)

        @pl.when(j2 > 0)
        def _():
            pltpu.make_async_copy(rowB, ACCv.at[dstb.at[jA]], ss).wait()
            pltpu.make_async_copy(exB, DENv.at[dstb.at[jA]], ss).wait()
        pltpu.make_async_copy(tab.at[srcb.at[jB]], rowB, gs).start()
        pltpu.make_async_copy(rowA, ACCv.at[dstb.at[jA]], ss).start(add=True)
        pltpu.make_async_copy(exA, DENv.at[dstb.at[jA]], ss).start(add=True)
        _edge_ex_phase(srcb, dstb, stab, dtab, exB, jB)
        pltpu.make_async_copy(tab.at[srcb.at[jB]], rowB, gs).wait()
        _scale_rows(rowB, exB, nreg, tail)
        pltpu.make_async_copy(rowA, ACCv.at[dstb.at[jA]], ss).wait()
        pltpu.make_async_copy(exA, DENv.at[dstb.at[jA]], ss).wait()

        @pl.when(j2 < npairs - 1)
        def _():
            pltpu.make_async_copy(tab.at[srcb.at[jA + 2]], rowA, gs).start()
        pltpu.make_async_copy(rowB, ACCv.at[dstb.at[jB]], ss).start(add=True)
        pltpu.make_async_copy(exB, DENv.at[dstb.at[jB]], ss).start(add=True)
        return carry
    lax.fori_loop(0, npairs, pair, 0)
    pltpu.make_async_copy(rowB, ACCv.at[dstb.at[0]], ss).wait()
    pltpu.make_async_copy(exB, DENv.at[dstb.at[0]], ss).wait()


def _sc1_body(h1t, s1, d1, src1, dst1,
              acc_out, den_out,
              srcb, dstb, stab, dtab, rowA, rowB, exA, exB, ACC, DEN,
              gs, ss):
    c = lax.axis_index("c")
    s = lax.axis_index("s")
    r0 = s * ROWS_PER
    r0a = s * RP1
    pltpu.sync_copy(src1.at[s], srcb)
    pltpu.sync_copy(dst1.at[s], dstb)
    _zero_buf(rowA, HID // 16)
    zv16 = jnp.zeros((16,), jnp.float32)
    for i in range(CHUNK // 16):
        exA[pl.ds(i * 16, 16)] = zv16
    for h in range(2):
        head = 2 * c + h
        pltpu.sync_copy(s1.at[head], stab.at[h])
        pltpu.sync_copy(d1.at[head], dtab.at[h])
        for b in range(RP1 // CHUNK):
            pltpu.sync_copy(rowA, ACC.at[h].at[pl.ds(r0a + b * CHUNK, CHUNK)])
        pltpu.sync_copy(rowA.at[pl.ds(0, RP1 % CHUNK)],
                        ACC.at[h].at[pl.ds(r0a + RP1 - RP1 % CHUNK,
                                           RP1 % CHUNK)])
        for b in range(ROWS_PER // CHUNK):
            pltpu.sync_copy(exA, DEN.at[h].at[pl.ds(r0 + b * CHUNK, CHUNK)])
    plsc.subcore_barrier()
    for h in range(2):
        head = 2 * c + h
        _sync_chunks(CH1, h1t.at[head], srcb, dstb,
                     stab.at[h], dtab.at[h], rowA, exA,
                     ACC.at[h], DEN.at[h], gs, HID // 16)
    plsc.subcore_barrier()
    for h in range(2):
        head = 2 * c + h
        pltpu.sync_copy(ACC.at[h].at[pl.ds(r0a, RP1)],
                        acc_out.at[head].at[pl.ds(r0a, RP1)])
        pltpu.sync_copy(DEN.at[h].at[pl.ds(r0, ROWS_PER)],
                        den_out.at[head].at[pl.ds(r0, ROWS_PER)])


def _sc2_body(h2t, s2, d2, src2, dst2,
              acc_out, den_out,
              srcb, dstb, stab, dtab, rowA, rowB, exA, exB, ACC, DEN,
              gs, ss):
    c = lax.axis_index("c")
    s = lax.axis_index("s")
    w = s * 2 + c
    r0 = s * ROWS_PER
    pltpu.sync_copy(src2.at[w], srcb)
    pltpu.sync_copy(dst2.at[w], dstb)
    pltpu.sync_copy(s2, stab)
    pltpu.sync_copy(d2, dtab)
    _zero_buf(rowA, D2P // 16, D2P % 16)
    zv16 = jnp.zeros((16,), jnp.float32)
    for i in range(CHUNK // 16):
        exA[pl.ds(i * 16, 16)] = zv16
    for b in range(ROWS_PER // CHUNK):
        pltpu.sync_copy(rowA, ACC.at[pl.ds(r0 + b * CHUNK, CHUNK)])
        pltpu.sync_copy(exA, DEN.at[pl.ds(r0 + b * CHUNK, CHUNK)])
    plsc.subcore_barrier()
    _pipe_chunks(CH2 // 2, h2t, srcb, dstb, stab, dtab,
                 rowA, rowB, exA, exB, ACC, DEN,
                 gs, ss, D2P // 16, D2P % 16)
    plsc.subcore_barrier()
    pltpu.sync_copy(ACC.at[pl.ds(r0, ROWS_PER)],
                    acc_out.at[c].at[pl.ds(r0, ROWS_PER)])
    pltpu.sync_copy(DEN.at[pl.ds(r0, ROWS_PER)],
                    den_out.at[c].at[pl.ds(r0, ROWS_PER)])


_sc1 = pl.kernel(
    _sc1_body,
    out_type=(jax.ShapeDtypeStruct((HEADS, NP1, HID), jnp.float32),
              jax.ShapeDtypeStruct((HEADS, NP), jnp.float32)),
    mesh=_MESH,
    scratch_types=[
        pltpu.VMEM((CH1, CHUNK), jnp.int32),
        pltpu.VMEM((CH1, CHUNK), jnp.int32),
        pltpu.VMEM((2, NP), jnp.float32),
        pltpu.VMEM((2, NP), jnp.float32),
        pltpu.VMEM((CHUNK, HID), jnp.float32),
        pltpu.VMEM((CHUNK, HID), jnp.float32),
        pltpu.VMEM((CHUNK,), jnp.float32),
        pltpu.VMEM((CHUNK,), jnp.float32),
        pltpu.VMEM_SHARED((2, NP1, HID), jnp.float32),
        pltpu.VMEM_SHARED((2, NP), jnp.float32),
        pltpu.SemaphoreType.DMA,
        pltpu.SemaphoreType.DMA,
    ],
    compiler_params=pltpu.CompilerParams(use_tc_tiling_on_sc=False,
                                         needs_layout_passes=False),
)

_sc2 = pl.kernel(
    _sc2_body,
    out_type=(jax.ShapeDtypeStruct((2, NP, D2P), jnp.float32),
              jax.ShapeDtypeStruct((2, NP), jnp.float32)),
    mesh=_MESH,
    scratch_types=[
        pltpu.VMEM((CH2, CHUNK), jnp.int32),
        pltpu.VMEM((CH2, CHUNK), jnp.int32),
        pltpu.VMEM((NP,), jnp.float32),
        pltpu.VMEM((NP,), jnp.float32),
        pltpu.VMEM((CHUNK, D2P), jnp.float32),
        pltpu.VMEM((CHUNK, D2P), jnp.float32),
        pltpu.VMEM((CHUNK,), jnp.float32),
        pltpu.VMEM((CHUNK,), jnp.float32),
        pltpu.VMEM_SHARED((NP, D2P), jnp.float32),
        pltpu.VMEM_SHARED((NP,), jnp.float32),
        pltpu.SemaphoreType.DMA,
        pltpu.SemaphoreType.DMA,
    ],
    compiler_params=pltpu.CompilerParams(use_tc_tiling_on_sc=False,
                                         needs_layout_passes=False),
)


def kernel(x, edge_index, W_att, a_att, W2, a2):
    src = edge_index[0]
    dst = edge_index[1]
    i32 = jnp.int32

    pad1 = TILES * CH1 * CHUNK - E
    src1 = jnp.concatenate([src, jnp.zeros((pad1,), i32)]).reshape(
        TILES, CH1, CHUNK)
    dst1 = jnp.concatenate([dst, jnp.full((pad1,), DUMP, i32)]).reshape(
        TILES, CH1, CHUNK)
    pad2 = WORKERS * CH2 * CHUNK - E
    src2 = jnp.concatenate([src, jnp.zeros((pad2,), i32)]).reshape(
        WORKERS, CH2, CHUNK)
    dst2 = jnp.concatenate([dst, jnp.full((pad2,), DUMP, i32)]).reshape(
        WORKERS, CH2, CHUNK)

    wcat = jnp.transpose(W_att, (1, 0, 2)).reshape(IN_F, HEADS * HID)
    scol = jnp.stack([W_att[h] @ a_att[h, :HID] for h in range(HEADS)], axis=1)
    dcol = jnp.stack([W_att[h] @ a_att[h, HID:] for h in range(HEADS)], axis=1)
    w1e = jnp.concatenate(
        [wcat, scol, dcol,
         jnp.zeros((IN_F, 256 - HEADS * HID - 2 * HEADS), jnp.float32)],
        axis=1)

    h1ext = _tc_matmul(x, w1e)
    h1 = jnp.pad(h1ext[:, :HEADS * HID], ((0, NP1 - N), (0, 0)))
    h1t = jnp.transpose(h1.reshape(NP1, HEADS, HID), (1, 0, 2))
    s1 = jnp.pad(h1ext[:, HEADS * HID:HEADS * HID + HEADS].T,
                 ((0, 0), (0, NP - N)))
    d1 = jnp.pad(h1ext[:, HEADS * HID + HEADS:HEADS * HID + 2 * HEADS].T,
                 ((0, 0), (0, NP - N)))

    acc1, den1 = _sc1(h1t, s1, d1, src1, dst1)

    acc_cat = jnp.transpose(acc1[:, :N, :], (1, 0, 2)).reshape(N, HEADS * HID)
    den_rep = jnp.repeat(den1[:, :N].T, HID, axis=1)
    w2e = jnp.concatenate(
        [W2, (W2 @ a2[:OUT_F])[:, None], (W2 @ a2[OUT_F:])[:, None],
         jnp.zeros((IN_F, 256 - OUT_F - 2), jnp.float32)], axis=1)
    h2ext = _tc_norm_matmul(acc_cat, den_rep, w2e)

    h2t = jnp.pad(h2ext[:, :OUT_F], ((0, NP - N), (0, 0)))
    s2 = jnp.pad(h2ext[:, OUT_F], (0, NP - N))
    d2 = jnp.pad(h2ext[:, OUT_F + 1], (0, NP - N))

    acc2, den2 = _sc2(h2t, s2, d2, src2, dst2)

    out = _tc_final(acc2[0, :N, :OUT_F], acc2[1, :N, :OUT_F],
                    den2[0, :N, None], den2[1, :N, None])
    return out


# speedup vs baseline: 1.1289x; 1.1289x over previous
import jax
import jax.numpy as jnp
from jax import lax
from jax.experimental import pallas as pl
from jax.experimental.pallas import tpu as pltpu
from jax.experimental.pallas import tpu_sc as plsc

N = 10000
E = 320000
IN_F = 128
HID = 32
HEADS = 4
OUT_F = 40
ALPHA = 0.2
EPS = 1e-16

NP = 10240
DUMP = N
CHUNK = 128
TILES = 16
WORKERS = 32
CH1 = 157
CH2 = 79
D2P = 48
NP1 = NP
ROWS_PER = NP // TILES
RP1 = NP1 // TILES


def _mm_body(x_ref, w_ref, o_ref):
    o_ref[...] = jnp.dot(x_ref[...], w_ref[...],
                         preferred_element_type=jnp.float32)


def _tc_matmul(x, w, bn=1000):
    n = x.shape[0]
    k = x.shape[1]
    m = w.shape[1]
    return pl.pallas_call(
        _mm_body,
        grid=(n // bn,),
        in_specs=[pl.BlockSpec((bn, k), lambda i: (i, 0)),
                  pl.BlockSpec((k, m), lambda i: (0, 0))],
        out_specs=pl.BlockSpec((bn, m), lambda i: (i, 0)),
        out_shape=jax.ShapeDtypeStruct((n, m), jnp.float32),
    )(x, w)


def _norm_mm_body(a_ref, d_ref, w_ref, o_ref):
    h = a_ref[...] / (d_ref[...] + EPS)
    h = jnp.where(h > 0, h, jnp.exp(jnp.minimum(h, 0.0)) - 1.0)
    o_ref[...] = jnp.dot(h, w_ref[...], preferred_element_type=jnp.float32)


def _tc_norm_matmul(acc, den, w, bn=1000):
    n, k = acc.shape
    m = w.shape[1]
    return pl.pallas_call(
        _norm_mm_body,
        grid=(n // bn,),
        in_specs=[pl.BlockSpec((bn, k), lambda i: (i, 0)),
                  pl.BlockSpec((bn, k), lambda i: (i, 0)),
                  pl.BlockSpec((k, m), lambda i: (0, 0))],
        out_specs=pl.BlockSpec((bn, m), lambda i: (i, 0)),
        out_shape=jax.ShapeDtypeStruct((n, m), jnp.float32),
    )(acc, den, w)


def _final_body(a0_ref, a1_ref, d0_ref, d1_ref, o_ref):
    den = d0_ref[...] + d1_ref[...] + EPS
    o = (a0_ref[...] + a1_ref[...]) / den
    m = jnp.max(o, axis=1, keepdims=True)
    o = o - m
    lse = jnp.log(jnp.sum(jnp.exp(o), axis=1, keepdims=True))
    o_ref[...] = o - lse


def _tc_final(a0, a1, d0, d1, bn=1000):
    n, m = a0.shape
    return pl.pallas_call(
        _final_body,
        grid=(n // bn,),
        in_specs=[pl.BlockSpec((bn, m), lambda i: (i, 0)),
                  pl.BlockSpec((bn, m), lambda i: (i, 0)),
                  pl.BlockSpec((bn, 1), lambda i: (i, 0)),
                  pl.BlockSpec((bn, 1), lambda i: (i, 0))],
        out_specs=pl.BlockSpec((bn, m), lambda i: (i, 0)),
        out_shape=jax.ShapeDtypeStruct((n, m), jnp.float32),
    )(a0, a1, d0, d1)


_MESH = plsc.VectorSubcoreMesh(core_axis_name="c", subcore_axis_name="s")


def _edge_ex_phase(srcb, dstb, stab, dtab, exbuf, j):
    for i in range(CHUNK // 16):
        si = srcb[j, pl.ds(i * 16, 16)]
        di = dstb[j, pl.ds(i * 16, 16)]
        sval = plsc.load_gather(stab, [si])
        dval = plsc.load_gather(dtab, [di])
        e = sval + dval
        e = jnp.maximum(e, ALPHA * e)
        exbuf[pl.ds(i * 16, 16)] = jnp.exp(e)


def _scale_rows(rowbuf, exbuf, nreg, tail=0, unroll=4):
    lane = lax.iota(jnp.int32, 16)
    tmask = lane < tail
    tcol = lane + nreg * 16

    def body(eu, carry):
        for u in range(unroll):
            e = eu * unroll + u
            ev = jnp.full((16,), 0, jnp.int32) + e
            ex = plsc.load_gather(exbuf, [ev])
            for k in range(nreg):
                v = rowbuf[e, pl.ds(k * 16, 16)]
                rowbuf[e, pl.ds(k * 16, 16)] = v * ex
            if tail:
                v = plsc.load_gather(rowbuf, [ev, tcol], mask=tmask)
                plsc.store_scatter(rowbuf, [ev, tcol], v * ex, mask=tmask)
        return carry
    lax.fori_loop(0, CHUNK // unroll, body, 0)


def _zero_buf(buf, ncol_reg, tail=0):
    zv = jnp.zeros((16,), jnp.float32)
    lane = lax.iota(jnp.int32, 16)
    tmask = lane < tail
    tcol = lane + ncol_reg * 16

    def zb(e, carry):
        for k in range(ncol_reg):
            buf[e, pl.ds(k * 16, 16)] = zv
        if tail:
            ev = jnp.full((16,), 0, jnp.int32) + e
            plsc.store_scatter(buf, [ev, tcol], zv, mask=tmask)
        return carry
    lax.fori_loop(0, CHUNK, zb, 0)


def _sync_chunks(nchunks, tab, srcb, dstb, stab, dtab, row, ex,
                 ACCv, DENv, gs, nreg, tail=0):
    def chunk_body(j, carry):
        cp = pltpu.async_copy(tab.at[srcb.at[j]], row, gs)
        _edge_ex_phase(srcb, dstb, stab, dtab, ex, j)
        cp.wait()
        _scale_rows(row, ex, nreg, tail)
        pltpu.sync_copy(row, ACCv.at[dstb.at[j]], add=True)
        pltpu.sync_copy(ex, DENv.at[dstb.at[j]], add=True)
        return carry
    lax.fori_loop(0, nchunks, chunk_body, 0)


def _pipe_chunks(npairs, tab, srcb, dstb, stab, dtab,
                 rowA, rowB, exA, exB, ACCv, DENv, gs, ss, nreg, tail=0):
    pltpu.make_async_copy(tab.at[srcb.at[0]], rowA, gs).start()

    def pair(j2, carry):
        jA = 2 * j2
        jB = jA + 1
        _edge_ex_phase(srcb, dstb, stab, dtab, exA, jA)
        pltpu.make_async_copy(tab.at[srcb.at[jA]], rowA, gs).wait()
        _scale_rows(rowA, exA, nreg, tail)

        @pl.when(j2 > 0)
        def _():
            pltpu.make_async_copy(rowB, ACCv.at[dstb.at[jA]], ss).wait()
            pltpu.make_async_copy(exB, DENv.at[dstb.at[jA]], ss).wait()
        pltpu.make_async_copy(tab.at[srcb.at[jB]], rowB, gs).start()
        pltpu.make_async_copy(rowA, ACCv.at[dstb.at[jA]], ss).start(add=True)
        pltpu.make_async_copy(exA, DENv.at[dstb.at[jA]], ss).start(add=True)
        _edge_ex_phase(srcb, dstb, stab, dtab, exB, jB)
        pltpu.make_async_copy(tab.at[srcb.at[jB]], rowB, gs).wait()
        _scale_rows(rowB, exB, nreg, tail)
        pltpu.make_async_copy(rowA, ACCv.at[dstb.at[jA]], ss).wait()
        pltpu.make_async_copy(exA, DENv.at[dstb.at[jA]], ss).wait()

        @pl.when(j2 < npairs - 1)
        def _():
            pltpu.make_async_copy(tab.at[srcb.at[jA + 2]], rowA, gs).start()
        pltpu.make_async_copy(rowB, ACCv.at[dstb.at[jB]], ss).start(add=True)
        pltpu.make_async_copy(exB, DENv.at[dstb.at[jB]], ss).start(add=True)
        return carry
    lax.fori_loop(0, npairs, pair, 0)
    pltpu.make_async_copy(rowB, ACCv.at[dstb.at[0]], ss).wait()
    pltpu.make_async_copy(exB, DENv.at[dstb.at[0]], ss).wait()


def _sc1_body(h1t, s1, d1, src1, dst1,
              acc_out, den_out,
              srcb, dstb, stab, dtab, rowA, rowB, exA, exB, ACC, DEN,
              gs, ss):
    c = lax.axis_index("c")
    s = lax.axis_index("s")
    r0 = s * ROWS_PER
    pltpu.sync_copy(src1.at[s], srcb)
    pltpu.sync_copy(dst1.at[s], dstb)
    _zero_buf(rowA, HID // 16)
    zv16 = jnp.zeros((16,), jnp.float32)
    for i in range(CHUNK // 16):
        exA[pl.ds(i * 16, 16)] = zv16
    for h in range(2):
        head = 2 * c + h
        pltpu.sync_copy(s1.at[head], stab.at[h])
        pltpu.sync_copy(d1.at[head], dtab.at[h])
        for b in range(ROWS_PER // CHUNK):
            pltpu.sync_copy(rowA, ACC.at[h].at[pl.ds(r0 + b * CHUNK, CHUNK)])
            pltpu.sync_copy(exA, DEN.at[h].at[pl.ds(r0 + b * CHUNK, CHUNK)])
    plsc.subcore_barrier()
    for h in range(2):
        head = 2 * c + h
        _sync_chunks(CH1, h1t.at[head], srcb, dstb,
                     stab.at[h], dtab.at[h], rowA, exA,
                     ACC.at[h], DEN.at[h], gs, HID // 16)
    plsc.subcore_barrier()
    for h in range(2):
        head = 2 * c + h
        pltpu.sync_copy(ACC.at[h].at[pl.ds(r0, ROWS_PER)],
                        acc_out.at[head].at[pl.ds(r0, ROWS_PER)])
        pltpu.sync_copy(DEN.at[h].at[pl.ds(r0, ROWS_PER)],
                        den_out.at[head].at[pl.ds(r0, ROWS_PER)])


def _sc2_body(h2t, s2, d2, src2, dst2,
              acc_out, den_out,
              srcb, dstb, stab, dtab, rowA, rowB, exA, exB, ACC, DEN,
              gs, ss):
    c = lax.axis_index("c")
    s = lax.axis_index("s")
    w = s * 2 + c
    r0 = s * ROWS_PER
    pltpu.sync_copy(src2.at[w], srcb)
    pltpu.sync_copy(dst2.at[w], dstb)
    pltpu.sync_copy(s2, stab)
    pltpu.sync_copy(d2, dtab)
    _zero_buf(rowA, D2P // 16)
    zv16 = jnp.zeros((16,), jnp.float32)
    for i in range(CHUNK // 16):
        exA[pl.ds(i * 16, 16)] = zv16
    for b in range(ROWS_PER // CHUNK):
        pltpu.sync_copy(rowA, ACC.at[pl.ds(r0 + b * CHUNK, CHUNK)])
        pltpu.sync_copy(exA, DEN.at[pl.ds(r0 + b * CHUNK, CHUNK)])
    plsc.subcore_barrier()
    _sync_chunks(CH2, h2t, srcb, dstb, stab, dtab, rowA, exA,
                 ACC, DEN, gs, D2P // 16)
    plsc.subcore_barrier()
    pltpu.sync_copy(ACC.at[pl.ds(r0, ROWS_PER)],
                    acc_out.at[c].at[pl.ds(r0, ROWS_PER)])
    pltpu.sync_copy(DEN.at[pl.ds(r0, ROWS_PER)],
                    den_out.at[c].at[pl.ds(r0, ROWS_PER)])


_sc1 = pl.kernel(
    _sc1_body,
    out_type=(jax.ShapeDtypeStruct((HEADS, NP1, HID), jnp.float32),
              jax.ShapeDtypeStruct((HEADS, NP), jnp.float32)),
    mesh=_MESH,
    scratch_types=[
        pltpu.VMEM((CH1, CHUNK), jnp.int32),
        pltpu.VMEM((CH1, CHUNK), jnp.int32),
        pltpu.VMEM((2, NP), jnp.float32),
        pltpu.VMEM((2, NP), jnp.float32),
        pltpu.VMEM((CHUNK, HID), jnp.float32),
        pltpu.VMEM((CHUNK, HID), jnp.float32),
        pltpu.VMEM((CHUNK,), jnp.float32),
        pltpu.VMEM((CHUNK,), jnp.float32),
        pltpu.VMEM_SHARED((2, NP1, HID), jnp.float32),
        pltpu.VMEM_SHARED((2, NP), jnp.float32),
        pltpu.SemaphoreType.DMA,
        pltpu.SemaphoreType.DMA,
    ],
    compiler_params=pltpu.CompilerParams(use_tc_tiling_on_sc=False,
                                         needs_layout_passes=False),
)

_sc2 = pl.kernel(
    _sc2_body,
    out_type=(jax.ShapeDtypeStruct((2, NP, D2P), jnp.float32),
              jax.ShapeDtypeStruct((2, NP), jnp.float32)),
    mesh=_MESH,
    scratch_types=[
        pltpu.VMEM((CH2, CHUNK), jnp.int32),
        pltpu.VMEM((CH2, CHUNK), jnp.int32),
        pltpu.VMEM((NP,), jnp.float32),
        pltpu.VMEM((NP,), jnp.float32),
        pltpu.VMEM((CHUNK, D2P), jnp.float32),
        pltpu.VMEM((CHUNK, D2P), jnp.float32),
        pltpu.VMEM((CHUNK,), jnp.float32),
        pltpu.VMEM((CHUNK,), jnp.float32),
        pltpu.VMEM_SHARED((NP, D2P), jnp.float32),
        pltpu.VMEM_SHARED((NP,), jnp.float32),
        pltpu.SemaphoreType.DMA,
        pltpu.SemaphoreType.DMA,
    ],
    compiler_params=pltpu.CompilerParams(use_tc_tiling_on_sc=False,
                                         needs_layout_passes=False),
)


def kernel(x, edge_index, W_att, a_att, W2, a2):
    src = edge_index[0]
    dst = edge_index[1]
    i32 = jnp.int32

    pad1 = TILES * CH1 * CHUNK - E
    src1 = jnp.concatenate([src, jnp.zeros((pad1,), i32)]).reshape(
        TILES, CH1, CHUNK)
    dst1 = jnp.concatenate([dst, jnp.full((pad1,), DUMP, i32)]).reshape(
        TILES, CH1, CHUNK)
    pad2 = WORKERS * CH2 * CHUNK - E
    src2 = jnp.concatenate([src, jnp.zeros((pad2,), i32)]).reshape(
        WORKERS, CH2, CHUNK)
    dst2 = jnp.concatenate([dst, jnp.full((pad2,), DUMP, i32)]).reshape(
        WORKERS, CH2, CHUNK)

    wcat = jnp.transpose(W_att, (1, 0, 2)).reshape(IN_F, HEADS * HID)
    scol = jnp.stack([W_att[h] @ a_att[h, :HID] for h in range(HEADS)], axis=1)
    dcol = jnp.stack([W_att[h] @ a_att[h, HID:] for h in range(HEADS)], axis=1)
    w1e = jnp.concatenate(
        [wcat, scol, dcol,
         jnp.zeros((IN_F, 256 - HEADS * HID - 2 * HEADS), jnp.float32)],
        axis=1)

    h1ext = _tc_matmul(x, w1e)
    h1 = jnp.pad(h1ext[:, :HEADS * HID], ((0, NP1 - N), (0, 0)))
    h1t = jnp.transpose(h1.reshape(NP1, HEADS, HID), (1, 0, 2))
    s1 = jnp.pad(h1ext[:, HEADS * HID:HEADS * HID + HEADS].T,
                 ((0, 0), (0, NP - N)))
    d1 = jnp.pad(h1ext[:, HEADS * HID + HEADS:HEADS * HID + 2 * HEADS].T,
                 ((0, 0), (0, NP - N)))

    acc1, den1 = _sc1(h1t, s1, d1, src1, dst1)

    acc_cat = jnp.transpose(acc1[:, :N, :], (1, 0, 2)).reshape(N, HEADS * HID)
    den_rep = jnp.repeat(den1[:, :N].T, HID, axis=1)
    w2e = jnp.concatenate(
        [W2, jnp.zeros((IN_F, D2P - OUT_F), jnp.float32),
         (W2 @ a2[:OUT_F])[:, None], (W2 @ a2[OUT_F:])[:, None],
         jnp.zeros((IN_F, 256 - D2P - 2), jnp.float32)], axis=1)
    h2ext = _tc_norm_matmul(acc_cat, den_rep, w2e)

    h2t = jnp.pad(h2ext[:, :D2P], ((0, NP - N), (0, 0)))
    s2 = jnp.pad(h2ext[:, D2P], (0, NP - N))
    d2 = jnp.pad(h2ext[:, D2P + 1], (0, NP - N))

    acc2, den2 = _sc2(h2t, s2, d2, src2, dst2)

    out = _tc_final(acc2[0, :N, :OUT_F], acc2[1, :N, :OUT_F],
                    den2[0, :N, None], den2[1, :N, None])
    return out
